# Initial kernel scaffold; baseline (speedup 1.0000x reference)
import jax
import jax.numpy as jnp
from jax.experimental import pallas as pl


def kernel(x, rms1_w, to_P_w, to_P_b, route_w, from_P_w, from_P_b, rms2_w, w12_w, w12_b, w3_w, w3_b):
    raise NotImplementedError("write your pallas kernel here")


# speedup vs baseline: 7.6810x; 7.6810x over previous
import functools

import jax
import jax.numpy as jnp
from jax.experimental import pallas as pl
from jax.experimental.pallas import tpu as pltpu

ALPHA = 16
EPS = 1e-6


def _mix_kernel(route_ref, fromp_ref, m_ref):
    r = route_ref[...]
    r = r - jnp.max(r, axis=-1, keepdims=True)
    e = jnp.exp(r)
    s = e / jnp.sum(e, axis=-1, keepdims=True)
    m_ref[...] = jax.lax.dot_general(
        s, fromp_ref[...], (((1,), (1,)), ((), ())),
        preferred_element_type=jnp.float32)


def _main_kernel(x_ref, rms1_ref, topw_ref, topb_ref, m_ref, frompb_ref,
                 rms2_ref, w1_ref, w2_ref, w1b_ref, w2b_ref, w3_ref, w3b_ref,
                 out_ref):
    xb = x_ref[...]
    ms = jnp.mean(xb * xb, axis=1, keepdims=True)
    h1 = xb * jax.lax.rsqrt(ms + EPS) * rms1_ref[...]
    u = jax.lax.dot_general(h1, topw_ref[...], (((1,), (1,)), ((), ())),
                            preferred_element_type=jnp.float32)
    u = u + topb_ref[...]
    absu = jnp.abs(u)
    work = absu
    for _ in range(ALPHA - 1):
        mx = jnp.max(work, axis=1, keepdims=True)
        work = jnp.where(work >= mx, -1.0, work)
    thr = jnp.max(work, axis=1, keepdims=True)
    t = jnp.where(absu >= thr, u, 0.0)
    x2 = xb + jax.lax.dot_general(t, m_ref[...], (((1,), (0,)), ((), ())),
                                  preferred_element_type=jnp.float32)
    x2 = x2 + frompb_ref[...]
    ms2 = jnp.mean(x2 * x2, axis=1, keepdims=True)
    h2 = x2 * jax.lax.rsqrt(ms2 + EPS) * rms2_ref[...]
    a = jax.lax.dot_general(h2, w1_ref[...], (((1,), (1,)), ((), ())),
                            preferred_element_type=jnp.float32) + w1b_ref[...]
    b = jax.lax.dot_general(h2, w2_ref[...], (((1,), (1,)), ((), ())),
                            preferred_element_type=jnp.float32) + w2b_ref[...]
    y = a * jax.nn.sigmoid(a) * b
    out = x2 + jax.lax.dot_general(y, w3_ref[...], (((1,), (1,)), ((), ())),
                                   preferred_element_type=jnp.float32)
    out_ref[...] = out + w3b_ref[...]


@functools.partial(jax.jit, static_argnames=("bn",))
def _run(x, rms1_w, to_P_w, to_P_b, route_w, from_P_w, from_P_b, rms2_w,
         w12_w, w12_b, w3_w, w3_b, bn=256):
    n, d = x.shape
    p = to_P_w.shape[0]
    hid2 = w12_w.shape[0]
    hid = hid2 // 2

    mix = pl.pallas_call(
        _mix_kernel,
        out_shape=jax.ShapeDtypeStruct((p, d), jnp.float32),
    )
    m = mix(route_w, from_P_w)

    w1 = w12_w[:hid]
    w2 = w12_w[hid:]
    w1b = w12_b[:hid].reshape(1, hid)
    w2b = w12_b[hid:].reshape(1, hid)

    grid = (n // bn,)
    full = lambda shape: pl.BlockSpec(shape, lambda i: (0, 0))
    out = pl.pallas_call(
        _main_kernel,
        grid=grid,
        in_specs=[
            pl.BlockSpec((bn, d), lambda i: (i, 0)),
            full((1, d)),
            full((p, d)),
            full((1, p)),
            full((p, d)),
            full((1, d)),
            full((1, d)),
            full((hid, d)),
            full((hid, d)),
            full((1, hid)),
            full((1, hid)),
            full((d, hid)),
            full((1, d)),
        ],
        out_specs=pl.BlockSpec((bn, d), lambda i: (i, 0)),
        out_shape=jax.ShapeDtypeStruct((n, d), jnp.float32),
        compiler_params=pltpu.CompilerParams(
            dimension_semantics=("arbitrary",),
        ),
    )(x, rms1_w.reshape(1, d), to_P_w, to_P_b.reshape(1, p), m,
      from_P_b.reshape(1, d), rms2_w.reshape(1, d), w1, w2, w1b, w2b,
      w3_w, w3_b.reshape(1, d))
    return out


def kernel(x, rms1_w, to_P_w, to_P_b, route_w, from_P_w, from_P_b, rms2_w,
           w12_w, w12_b, w3_w, w3_b):
    return _run(x, rms1_w, to_P_w, to_P_b, route_w, from_P_w, from_P_b,
                rms2_w, w12_w, w12_b, w3_w, w3_b)


# speedup vs baseline: 7.7326x; 1.0067x over previous
import functools

import jax
import jax.numpy as jnp
from jax.experimental import pallas as pl
from jax.experimental.pallas import tpu as pltpu

ALPHA = 16
EPS = 1e-6


def _mix_kernel(route_ref, fromp_ref, m_ref):
    r = route_ref[...]
    r = r - jnp.max(r, axis=-1, keepdims=True)
    e = jnp.exp(r)
    s = e / jnp.sum(e, axis=-1, keepdims=True)
    m_ref[...] = jax.lax.dot_general(
        s, fromp_ref[...], (((1,), (1,)), ((), ())),
        preferred_element_type=jnp.float32).astype(jnp.bfloat16)


def _main_kernel(x_ref, rms1_ref, topw_ref, topb_ref, m_ref, frompb_ref,
                 rms2_ref, w1_ref, w2_ref, w1b_ref, w2b_ref, w3_ref, w3b_ref,
                 out_ref):
    xb = x_ref[...]
    ms = jnp.mean(xb * xb, axis=1, keepdims=True)
    h1 = xb * jax.lax.rsqrt(ms + EPS) * rms1_ref[...]
    u = jax.lax.dot_general(h1, topw_ref[...], (((1,), (1,)), ((), ())),
                            preferred_element_type=jnp.float32)
    u = u + topb_ref[...]
    absu = jnp.abs(u)
    work = absu
    for _ in range(ALPHA - 1):
        mx = jnp.max(work, axis=1, keepdims=True)
        work = jnp.where(work >= mx, -1.0, work)
    thr = jnp.max(work, axis=1, keepdims=True)
    t = jnp.where(absu >= thr, u, 0.0).astype(jnp.bfloat16)
    x2 = xb + jax.lax.dot_general(t, m_ref[...], (((1,), (0,)), ((), ())),
                                  preferred_element_type=jnp.float32)
    x2 = x2 + frompb_ref[...]
    ms2 = jnp.mean(x2 * x2, axis=1, keepdims=True)
    h2 = (x2 * jax.lax.rsqrt(ms2 + EPS) * rms2_ref[...]).astype(jnp.bfloat16)
    a = jax.lax.dot_general(h2, w1_ref[...], (((1,), (1,)), ((), ())),
                            preferred_element_type=jnp.float32) + w1b_ref[...]
    b = jax.lax.dot_general(h2, w2_ref[...], (((1,), (1,)), ((), ())),
                            preferred_element_type=jnp.float32) + w2b_ref[...]
    y = (a * jax.nn.sigmoid(a) * b).astype(jnp.bfloat16)
    out = x2 + jax.lax.dot_general(y, w3_ref[...], (((1,), (1,)), ((), ())),
                                   preferred_element_type=jnp.float32)
    out_ref[...] = out + w3b_ref[...]


@functools.partial(jax.jit, static_argnames=("bn",))
def _run(x, rms1_w, to_P_w, to_P_b, route_w, from_P_w, from_P_b, rms2_w,
         w12_w, w12_b, w3_w, w3_b, bn=256):
    n, d = x.shape
    p = to_P_w.shape[0]
    hid2 = w12_w.shape[0]
    hid = hid2 // 2

    mix = pl.pallas_call(
        _mix_kernel,
        out_shape=jax.ShapeDtypeStruct((p, d), jnp.bfloat16),
    )
    m = mix(route_w, from_P_w)

    w1 = w12_w[:hid].astype(jnp.bfloat16)
    w2 = w12_w[hid:].astype(jnp.bfloat16)
    w3t = w3_w.astype(jnp.bfloat16)
    w1b = w12_b[:hid].reshape(1, hid)
    w2b = w12_b[hid:].reshape(1, hid)

    grid = (n // bn,)
    full = lambda shape: pl.BlockSpec(shape, lambda i: (0, 0))
    out = pl.pallas_call(
        _main_kernel,
        grid=grid,
        in_specs=[
            pl.BlockSpec((bn, d), lambda i: (i, 0)),
            full((1, d)),
            full((p, d)),
            full((1, p)),
            full((p, d)),
            full((1, d)),
            full((1, d)),
            full((hid, d)),
            full((hid, d)),
            full((1, hid)),
            full((1, hid)),
            full((d, hid)),
            full((1, d)),
        ],
        out_specs=pl.BlockSpec((bn, d), lambda i: (i, 0)),
        out_shape=jax.ShapeDtypeStruct((n, d), jnp.float32),
        compiler_params=pltpu.CompilerParams(
            dimension_semantics=("arbitrary",),
        ),
    )(x, rms1_w.reshape(1, d), to_P_w, to_P_b.reshape(1, p), m,
      from_P_b.reshape(1, d), rms2_w.reshape(1, d), w1, w2, w1b, w2b,
      w3t, w3_b.reshape(1, d))
    return out


def kernel(x, rms1_w, to_P_w, to_P_b, route_w, from_P_w, from_P_b, rms2_w,
           w12_w, w12_b, w3_w, w3_b):
    return _run(x, rms1_w, to_P_w, to_P_b, route_w, from_P_w, from_P_b,
                rms2_w, w12_w, w12_b, w3_w, w3_b)


# speedup vs baseline: 8.0733x; 1.0441x over previous
import functools

import jax
import jax.numpy as jnp
from jax.experimental import pallas as pl
from jax.experimental.pallas import tpu as pltpu

ALPHA = 16
EPS = 1e-6


def _mix_kernel(route_ref, fromp_ref, m_ref):
    r = route_ref[...]
    r = r - jnp.max(r, axis=-1, keepdims=True)
    e = jnp.exp(r)
    s = e / jnp.sum(e, axis=-1, keepdims=True)
    m_ref[...] = jax.lax.dot_general(
        s, fromp_ref[...], (((1,), (1,)), ((), ())),
        preferred_element_type=jnp.float32).astype(jnp.bfloat16)


def _make_main(bn):
    def _main_kernel(x_ref, rms1_ref, topw_ref, topb_ref, m_ref, frompb_ref,
                     rms2_ref, w1_ref, w2_ref, w1b_ref, w2b_ref, w3_ref,
                     w3b_ref, out_ref, xs_ref, ts_ref):
        i = pl.program_id(0)
        rd = (i + 1) & 1
        wr = i & 1

        xb2 = xs_ref[pl.ds(rd * bn, bn), :]
        t = ts_ref[pl.ds(rd * bn, bn), :]
        x2 = xb2 + jax.lax.dot_general(t, m_ref[...], (((1,), (0,)), ((), ())),
                                       preferred_element_type=jnp.float32)
        x2 = x2 + frompb_ref[...]
        ms2 = jnp.mean(x2 * x2, axis=1, keepdims=True)
        h2 = (x2 * jax.lax.rsqrt(ms2 + EPS) * rms2_ref[...]).astype(jnp.bfloat16)
        a = jax.lax.dot_general(h2, w1_ref[...], (((1,), (1,)), ((), ())),
                                preferred_element_type=jnp.float32) + w1b_ref[...]
        b = jax.lax.dot_general(h2, w2_ref[...], (((1,), (1,)), ((), ())),
                                preferred_element_type=jnp.float32) + w2b_ref[...]
        y = (a * jax.nn.sigmoid(a) * b).astype(jnp.bfloat16)
        out = x2 + jax.lax.dot_general(y, w3_ref[...], (((1,), (1,)), ((), ())),
                                       preferred_element_type=jnp.float32)
        out_ref[...] = out + w3b_ref[...]

        xb = x_ref[...]
        ms = jnp.mean(xb * xb, axis=1, keepdims=True)
        h1 = xb * jax.lax.rsqrt(ms + EPS) * rms1_ref[...]
        u = jax.lax.dot_general(h1, topw_ref[...], (((1,), (1,)), ((), ())),
                                preferred_element_type=jnp.float32)
        u = u + topb_ref[...]
        absu = jnp.abs(u)
        work = absu
        for _ in range(ALPHA - 1):
            mx = jnp.max(work, axis=1, keepdims=True)
            work = jnp.where(work >= mx, -1.0, work)
        thr = jnp.max(work, axis=1, keepdims=True)
        t_new = jnp.where(absu >= thr, u, 0.0).astype(jnp.bfloat16)
        xs_ref[pl.ds(wr * bn, bn), :] = xb
        ts_ref[pl.ds(wr * bn, bn), :] = t_new

    return _main_kernel


@functools.partial(jax.jit, static_argnames=("bn",))
def _run(x, rms1_w, to_P_w, to_P_b, route_w, from_P_w, from_P_b, rms2_w,
         w12_w, w12_b, w3_w, w3_b, bn=256):
    n, d = x.shape
    p = to_P_w.shape[0]
    hid2 = w12_w.shape[0]
    hid = hid2 // 2

    mix = pl.pallas_call(
        _mix_kernel,
        out_shape=jax.ShapeDtypeStruct((p, d), jnp.bfloat16),
    )
    m = mix(route_w, from_P_w)

    w1 = w12_w[:hid].astype(jnp.bfloat16)
    w2 = w12_w[hid:].astype(jnp.bfloat16)
    w3t = w3_w.astype(jnp.bfloat16)
    w1b = w12_b[:hid].reshape(1, hid)
    w2b = w12_b[hid:].reshape(1, hid)

    nb = n // bn
    last = nb - 1
    grid = (nb + 1,)
    full = lambda shape: pl.BlockSpec(shape, lambda i: (0, 0))
    out = pl.pallas_call(
        _make_main(bn),
        grid=grid,
        in_specs=[
            pl.BlockSpec((bn, d), lambda i: (jnp.minimum(i, last), 0)),
            full((1, d)),
            full((p, d)),
            full((1, p)),
            full((p, d)),
            full((1, d)),
            full((1, d)),
            full((hid, d)),
            full((hid, d)),
            full((1, hid)),
            full((1, hid)),
            full((d, hid)),
            full((1, d)),
        ],
        out_specs=pl.BlockSpec((bn, d), lambda i: (jnp.maximum(i - 1, 0), 0)),
        out_shape=jax.ShapeDtypeStruct((n, d), jnp.float32),
        scratch_shapes=[
            pltpu.VMEM((2 * bn, d), jnp.float32),
            pltpu.VMEM((2 * bn, p), jnp.bfloat16),
        ],
        compiler_params=pltpu.CompilerParams(
            dimension_semantics=("arbitrary",),
        ),
    )(x, rms1_w.reshape(1, d), to_P_w, to_P_b.reshape(1, p), m,
      from_P_b.reshape(1, d), rms2_w.reshape(1, d), w1, w2, w1b, w2b,
      w3t, w3_b.reshape(1, d))
    return out


def kernel(x, rms1_w, to_P_w, to_P_b, route_w, from_P_w, from_P_b, rms2_w,
           w12_w, w12_b, w3_w, w3_b):
    return _run(x, rms1_w, to_P_w, to_P_b, route_w, from_P_w, from_P_b,
                rms2_w, w12_w, w12_b, w3_w, w3_b)
